# baseline (speedup 1.0000x reference)
import functools

import jax
import jax.numpy as jnp
from jax.experimental import pallas as pl
from jax.experimental.pallas import tpu as pltpu

N = 10000
E = 160000


def _dense_kernel(x_ref, w_ref, ast_ref, h_ref, scal_ref):
    x = x_ref[...]
    w = w_ref[0]
    h = jax.lax.dot_general(x, w, (((1,), (0,)), ((), ())),
                            preferred_element_type=jnp.float32)
    h_ref[0] = h
    scal_ref[0] = jax.lax.dot_general(h, ast_ref[0], (((1,), (1,)), ((), ())),
                                      preferred_element_type=jnp.float32)


def _dense_heads(x, W, a, bn=400):
    H, Din, u = W.shape
    nb = N // bn
    a2 = a[:, :, 0].reshape(H, 2, u)
    ast = jnp.concatenate([a2, jnp.zeros((H, 6, u), jnp.float32)], axis=1)
    h, scal = pl.pallas_call(
        _dense_kernel,
        grid=(H, nb),
        in_specs=[
            pl.BlockSpec((bn, Din), lambda i, n: (n, 0)),
            pl.BlockSpec((1, Din, u), lambda i, n: (i, 0, 0)),
            pl.BlockSpec((1, 8, u), lambda i, n: (i, 0, 0)),
        ],
        out_specs=[
            pl.BlockSpec((1, bn, u), lambda i, n: (i, n, 0)),
            pl.BlockSpec((1, bn, 8), lambda i, n: (i, n, 0)),
        ],
        out_shape=[
            jax.ShapeDtypeStruct((H, N, u), jnp.float32),
            jax.ShapeDtypeStruct((H, N, 8), jnp.float32),
        ],
    )(x, W, ast)
    return h, scal[:, :, 0], scal[:, :, 1]


def _gat_layer(x, dst, src, W, a, merge):
    h, sd, ss = _dense_heads(x, W, a)
    H = W.shape[0]
    heads = []
    for i in range(H):
        t = sd[i][dst] + ss[i][src]
        t = jnp.where(t > 0, t, 0.2 * t)
        s = jnp.exp(jnp.clip(t, -2.0, 2.0))
        denom = jax.ops.segment_sum(s, dst, num_segments=N)
        num = jax.ops.segment_sum(s[:, None] * h[i][src], dst, num_segments=N)
        heads.append(num / (denom[:, None] + 1e-9))
    if merge == "concat":
        out = jnp.concatenate(heads, axis=-1)
    else:
        out = jnp.mean(jnp.stack(heads, axis=0), axis=0)
    return jax.nn.relu(out)


def kernel(node_states, edges, training, W1, a1, W2, a2, W3, a3):
    dst = edges[:, 0]
    src = edges[:, 1]
    x = _gat_layer(node_states, dst, src, W1, a1, "concat")
    x = _gat_layer(x, dst, src, W2, a2, "concat")
    return _gat_layer(x, dst, src, W3, a3, "avg")


# speedup vs baseline: 7.8915x; 7.8915x over previous
import dataclasses
import functools

import jax
import jax.numpy as jnp
from jax import lax
from jax.experimental import pallas as pl
from jax.experimental.pallas import tpu as pltpu
from jax.experimental.pallas import tpu_sc as plsc

N = 10000
E = 160000
NA = 10240
RPT = NA // 16
WE = 128
WPT = 79
EPT = WPT * WE
E_SC = 16 * EPT


def _dense_kernel(nchunks, x_ref, w_ref, ast_ref, *out_refs):
    x = x_ref[...]
    w = w_ref[0]
    h = jax.lax.dot_general(x, w, (((1,), (0,)), ((), ())),
                            preferred_element_type=jnp.float32)
    for k in range(nchunks):
        out_refs[k][0] = h[:, k * 128:(k + 1) * 128]
    out_refs[nchunks][0] = jax.lax.dot_general(
        h, ast_ref[0], (((1,), (1,)), ((), ())),
        preferred_element_type=jnp.float32)


def _dense_heads(x, W, a, bn=400):
    H, Din, u = W.shape
    nchunks = u // 128
    nb = N // bn
    a2 = a[:, :, 0].reshape(H, 2, u)
    ast = jnp.concatenate([a2, jnp.zeros((H, 6, u), jnp.float32)], axis=1)
    outs = pl.pallas_call(
        functools.partial(_dense_kernel, nchunks),
        grid=(H, nb),
        in_specs=[
            pl.BlockSpec((bn, Din), lambda i, n: (n, 0)),
            pl.BlockSpec((1, Din, u), lambda i, n: (i, 0, 0)),
            pl.BlockSpec((1, 8, u), lambda i, n: (i, 0, 0)),
        ],
        out_specs=[pl.BlockSpec((1, bn, 128), lambda i, n: (i, n, 0))
                   for _ in range(nchunks)]
                  + [pl.BlockSpec((1, bn, 8), lambda i, n: (i, n, 0))],
        out_shape=[jax.ShapeDtypeStruct((H, N, 128), jnp.float32)
                   for _ in range(nchunks)]
                  + [jax.ShapeDtypeStruct((H, N, 8), jnp.float32)],
    )(x, W, ast)
    h_chunks, scal = outs[:nchunks], outs[nchunks]
    return h_chunks, scal[:, :, 0], scal[:, :, 1]


def _sc_gat_body(relu, pair_base,
                 h_hbm, sdp_hbm, ss_hbm, dst_hbm, src_hbm, out_hbm,
                 sdp_loc, ss_loc, rows, sw, dloc, dstw, srcw, gidx, zvec,
                 acc_sh, den_sh):
    c = lax.axis_index("c")
    s = lax.axis_index("s")
    head = pair_base + c
    zero16 = jnp.zeros((16,), jnp.float32)

    pltpu.sync_copy(sdp_hbm.at[pl.ds(head * NA, NA)], sdp_loc)
    pltpu.sync_copy(ss_hbm.at[pl.ds(head * N, N)], ss_loc)

    @pl.loop(0, 128)
    def _zr(r):
        for j in range(8):
            rows[r, pl.ds(j * 16, 16)] = zero16

    @pl.loop(0, RPT // 16)
    def _zv(i):
        zvec[pl.ds(i * 16, 16)] = zero16

    for k in range(RPT // 128):
        pltpu.sync_copy(rows, acc_sh.at[pl.ds(s * RPT + k * 128, 128)])
    pltpu.sync_copy(zvec, den_sh.at[pl.ds(s * RPT, RPT)])
    plsc.subcore_barrier()

    @pl.loop(0, WPT)
    def _win(w):
        base = s * EPT + w * WE
        pltpu.sync_copy(dst_hbm.at[pl.ds(base, WE)], dstw)
        pltpu.sync_copy(src_hbm.at[pl.ds(base, WE)], srcw)
        for j in range(8):
            di = dstw[pl.ds(j * 16, 16)]
            si = srcw[pl.ds(j * 16, 16)]
            t = plsc.load_gather(sdp_loc, [di]) + plsc.load_gather(ss_loc, [si])
            t = jnp.where(t > 0, t, 0.2 * t)
            t = jnp.minimum(jnp.maximum(t, -2.0), 2.0)
            sw[pl.ds(j * 16, 16)] = jnp.exp(t)
            gidx[pl.ds(j * 16, 16)] = si + head * N
        pltpu.sync_copy(sw, den_sh.at[dstw], add=True)
        pltpu.sync_copy(h_hbm.at[gidx], rows)
        @pl.loop(0, WE)
        def _scale(r):
            sv = plsc.load_gather(sw, [jnp.full((16,), r, jnp.int32)])
            for j in range(8):
                rows[r, pl.ds(j * 16, 16)] = rows[r, pl.ds(j * 16, 16)] * sv
        pltpu.sync_copy(rows, acc_sh.at[dstw], add=True)

    plsc.subcore_barrier()

    for k in range(RPT // 128):
        row0 = s * RPT + k * 128
        pltpu.sync_copy(acc_sh.at[pl.ds(row0, 128)], rows)
        pltpu.sync_copy(den_sh.at[pl.ds(row0, 128)], dloc)

        @pl.loop(0, 128)
        def _norm(r):
            dv = plsc.load_gather(dloc, [jnp.full((16,), r, jnp.int32)])
            scale = 1.0 / (dv + 1e-9)
            for j in range(8):
                v = rows[r, pl.ds(j * 16, 16)] * scale
                if relu:
                    v = jnp.maximum(v, 0.0)
                rows[r, pl.ds(j * 16, 16)] = v

        pltpu.sync_copy(rows, out_hbm.at[pl.ds(c * NA + row0, 128)])


def _sc_compiler_params():
    cp = pltpu.CompilerParams()
    if "needs_layout_passes" in pltpu.CompilerParams.__dataclass_fields__:
        cp = dataclasses.replace(cp, needs_layout_passes=False)
    return cp


def _sc_pair(h_flat, sdp_flat, ss_flat, dst_sc, src_sc, pair_base, relu):
    mesh = plsc.VectorSubcoreMesh(core_axis_name="c", subcore_axis_name="s")
    body = functools.partial(_sc_gat_body, relu, pair_base)
    run = pl.kernel(
        body,
        out_type=jax.ShapeDtypeStruct((2 * NA, 128), jnp.float32),
        mesh=mesh,
        scratch_types=[
            pltpu.VMEM((NA,), jnp.float32),
            pltpu.VMEM((N,), jnp.float32),
            pltpu.VMEM((128, 128), jnp.float32),
            pltpu.VMEM((WE,), jnp.float32),
            pltpu.VMEM((128,), jnp.float32),
            pltpu.VMEM((WE,), jnp.int32),
            pltpu.VMEM((WE,), jnp.int32),
            pltpu.VMEM((WE,), jnp.int32),
            pltpu.VMEM((RPT,), jnp.float32),
            pltpu.VMEM_SHARED((NA, 128), jnp.float32),
            pltpu.VMEM_SHARED((NA,), jnp.float32),
        ],
        compiler_params=_sc_compiler_params(),
    )
    return run(h_flat, sdp_flat, ss_flat, dst_sc, src_sc)


def _gat_layer(x, dst_sc, src_sc, W, a, merge):
    H = W.shape[0]
    h_chunks, sd, ss = _dense_heads(x, W, a)
    sdp = jnp.pad(sd, ((0, 0), (0, NA - N))).reshape(-1)
    ssf = ss.reshape(-1)
    outs = {}
    for p in range(H // 2):
        for ci, hh in enumerate(h_chunks):
            o = _sc_pair(hh.reshape(H * N, 128), sdp, ssf, dst_sc, src_sc,
                         2 * p, merge == "concat")
            o = o.reshape(2, NA, 128)
            outs[(2 * p, ci)] = o[0, :N]
            outs[(2 * p + 1, ci)] = o[1, :N]
    if merge == "concat":
        cols = [outs[(hd, ci)] for hd in range(H) for ci in range(len(h_chunks))]
        return jnp.concatenate(cols, axis=1)
    acc = outs[(0, 0)]
    for hd in range(1, H):
        acc = acc + outs[(hd, 0)]
    return jax.nn.relu(acc / H)


def kernel(node_states, edges, training, W1, a1, W2, a2, W3, a3):
    dst = edges[:, 0]
    src = edges[:, 1]
    pad = E_SC - E
    e = jnp.arange(pad, dtype=jnp.int32)
    dst_sc = jnp.concatenate([dst, N + (e % 64)])
    src_sc = jnp.concatenate([src, e % N])
    x = _gat_layer(node_states, dst_sc, src_sc, W1, a1, "concat")
    x = _gat_layer(x, dst_sc, src_sc, W2, a2, "concat")
    return _gat_layer(x, dst_sc, src_sc, W3, a3, "avg")
